# baseline (speedup 1.0000x reference)
import functools

import jax
import jax.numpy as jnp
from jax import lax
from jax.experimental import pallas as pl
from jax.experimental.pallas import tpu as pltpu
from jax.experimental.pallas import tpu_sc as plsc

NUM_ENTITIES = 100000
NUM_RELATIONS = 474
D = 200
B = 2048
A = 256
RPAD = 512
XPAD = 256
HUGE = 1e31

NC, NS = 2, 16
NW = NC * NS
ROWS_W = B // NW
CHUNK = 128
_BB = 256

_sc_mesh = functools.partial(
    plsc.VectorSubcoreMesh, core_axis_name="c", subcore_axis_name="s",
    num_cores=NC, num_subcores=NS)

_SC_PARAMS = pltpu.CompilerParams(use_tc_tiling_on_sc=False,
                                  needs_layout_passes=False)


def _wid():
    return lax.axis_index("s") * NC + lax.axis_index("c")


def _repack_body(x_ref, a_ref):
    a_ref[...] = jnp.concatenate(
        [x_ref[...], jnp.zeros((x_ref.shape[0], XPAD - D), jnp.float32)],
        axis=1)


_RB = 2000


def _repack(ent):
    nblk = NUM_ENTITIES // _RB
    return pl.pallas_call(
        _repack_body,
        grid=(nblk,),
        in_specs=[pl.BlockSpec((_RB, D), lambda i: (i, 0))],
        out_specs=pl.BlockSpec((_RB, XPAD), lambda i: (i, 0)),
        out_shape=jax.ShapeDtypeStruct((NUM_ENTITIES, XPAD), jnp.float32),
    )(ent)


def _gather_e_body(ta_hbm, ce_hbm, ea_out, eidx, arows, sem_a):
    base = _wid() * ROWS_W
    pltpu.sync_copy(ce_hbm.at[pl.ds(base, ROWS_W)], eidx)
    pltpu.async_copy(ta_hbm.at[eidx], arows, sem_a).wait()
    pltpu.sync_copy(arows, ea_out.at[pl.ds(base, ROWS_W)])


@functools.cache
def _gather_e():
    return pl.kernel(
        _gather_e_body,
        out_type=jax.ShapeDtypeStruct((B, XPAD), jnp.float32),
        mesh=_sc_mesh(),
        scratch_types=[
            pltpu.VMEM((ROWS_W,), jnp.int32),
            pltpu.VMEM((ROWS_W, XPAD), jnp.float32),
            pltpu.SemaphoreType.DMA,
        ],
        compiler_params=_SC_PARAMS,
    )


def _policy_body(ea_ref, h_ref, qr_ref, w1e_ref, w1h_ref,
                 w1q_ref, b1_ref, relp_ref, w2a_ref, w2e_ref, b2a_ref,
                 b2e_ref, relt_ref, x2e_ref, rl_ref):
    f32 = jnp.float32
    onehot_t = (lax.broadcasted_iota(jnp.int32, (RPAD, _BB), 0)
                == qr_ref[0]).astype(f32)
    q = lax.dot_general(onehot_t, relp_ref[...], (((0,), (0,)), ((), ())),
                        preferred_element_type=f32)
    h1 = (jnp.dot(ea_ref[...], w1e_ref[...], preferred_element_type=f32)
          + jnp.dot(h_ref[...], w1h_ref[...], preferred_element_type=f32)
          + jnp.dot(q, w1q_ref[...], preferred_element_type=f32)
          + b1_ref[...])
    h1 = jnp.maximum(h1, 0.0)
    x2a = jnp.dot(h1, w2a_ref[...], preferred_element_type=f32) + b2a_ref[...]
    x2e = jnp.dot(h1, w2e_ref[...], preferred_element_type=f32) + b2e_ref[...]
    x2e_ref[...] = x2e
    rl_ref[...] = jnp.dot(x2a, relt_ref[...], preferred_element_type=f32)


def _policy(ea, h, qr3, w1e, w1h, w1q, b1, relp, w2a, w2e, b2a, b2e, relt):
    nblk = B // _BB
    row_spec = lambda cols: pl.BlockSpec((_BB, cols), lambda i: (i, 0))
    full = lambda shape: pl.BlockSpec(shape, lambda i: tuple(0 for _ in shape))
    return pl.pallas_call(
        _policy_body,
        grid=(nblk,),
        in_specs=[row_spec(XPAD), row_spec(D),
                  pl.BlockSpec((1, 1, _BB), lambda i: (i, 0, 0)),
                  full((XPAD, 2 * D)), full((D, 2 * D)),
                  full((D, 2 * D)), full((1, 2 * D)), full((RPAD, D)),
                  full((2 * D, D)), full((2 * D, XPAD)),
                  full((1, D)), full((1, XPAD)), full((D, RPAD))],
        out_specs=[row_spec(XPAD), row_spec(RPAD)],
        out_shape=[jax.ShapeDtypeStruct((B, XPAD), jnp.float32),
                   jax.ShapeDtypeStruct((B, RPAD), jnp.float32)],
    )(ea, h, qr3, w1e, w1h, w1q, b1, relp, w2a, w2e, b2a, b2e, relt)


def _logits_sc_body(ta_hbm, es_hbm, rs_hbm, rl_hbm, x2_hbm, out_hbm,
                    es_v, rs_v, rl_v, x_v, rows0_v, lg_v,
                    sem0, sem1, sem_es, sem_rs, sem_out):
    base = _wid() * ROWS_W
    pltpu.sync_copy(rl_hbm.at[pl.ds(base, ROWS_W)], rl_v)
    pltpu.sync_copy(x2_hbm.at[pl.ds(base, ROWS_W)], x_v)
    lane = lax.iota(jnp.int32, 16)
    zero = jnp.zeros((16,), jnp.float32)

    def copy_es(i, p, sem=None):
        for c in range(2):
            src = es_hbm.at[base + i, pl.ds(c * CHUNK, CHUNK)]
            if sem is None:
                pltpu.sync_copy(src, es_v.at[p, c])
            else:
                pltpu.async_copy(src, es_v.at[p, c], sem)

    def wait_es(i, p, sem):
        for c in range(2):
            src = es_hbm.at[base + i, pl.ds(c * CHUNK, CHUNK)]
            pltpu.make_async_copy(src, es_v.at[p, c], sem).wait()

    def gather_chunk(c, slot, p, sem):
        idx = es_v.at[p, c]
        return pltpu.async_copy(ta_hbm.at[idx], rows0_v.at[slot], sem)

    def wait_chunk(c, slot, p, sem):
        idx = es_v.at[p, c]
        pltpu.make_async_copy(ta_hbm.at[idx], rows0_v.at[slot], sem).wait()

    _gd = lax.GatherDimensionNumbers(offset_dims=(), collapsed_slice_dims=(0,),
                                     start_index_map=(0,))

    def perm_xor(v, s):
        return lax.gather(v, (lane ^ (1 << s))[:, None], _gd, (1,),
                          mode=lax.GatherScatterMode.PROMISE_IN_BOUNDS)

    def reduce16(vs):
        for s in range(4):
            m = (lane & (1 << s)) != 0
            nxt = []
            for k in range(len(vs) // 2):
                a, b = vs[2 * k], vs[2 * k + 1]
                t1 = jnp.where(m, b, a)
                t2 = jnp.where(m, a, b)
                nxt.append(t1 + perm_xor(t2, s))
            vs = nxt
        return vs[0]

    def compute_chunk(i, c, slot, p):
        r0 = rows0_v.at[slot]
        ivec = jnp.full((16,), i, jnp.int32)
        xs = [x_v[i, pl.ds(16 * j, 16)] for j in range(13)]

        def blk_body(blk, carry2):
            partials = []
            for t in range(16):
                acc = zero
                for j in range(13):
                    v = r0[blk * 16 + t, pl.ds(16 * j, 16)]
                    acc = acc + v * xs[j]
                partials.append(acc)
            acc = reduce16(partials)
            ridx = rs_v[p, pl.ds(c * CHUNK + blk * 16, 16)]
            rvals = plsc.load_gather(rl_v, [ivec, ridx])
            lg_v[p, pl.ds(c * CHUNK + blk * 16, 16)] = acc + rvals
            return carry2

        lax.fori_loop(0, CHUNK // 16, blk_body, 0)

    copy_es(0, 0)
    pltpu.sync_copy(rs_hbm.at[base], rs_v.at[0])
    gather_chunk(0, 0, 0, sem0)

    def row_body(i, carry):
        p = lax.rem(i, 2)
        pn = lax.rem(i + 1, 2)
        not_last = i < ROWS_W - 1
        gather_chunk(1, 1, p, sem1)

        @pl.when(not_last)
        def _prefetch_meta():
            copy_es(i + 1, pn, sem_es)
            pltpu.async_copy(rs_hbm.at[base + i + 1], rs_v.at[pn], sem_rs)

        wait_chunk(0, 0, p, sem0)
        compute_chunk(i, 0, 0, p)

        @pl.when(not_last)
        def _issue_next_row():
            wait_es(i + 1, pn, sem_es)
            gather_chunk(0, 0, pn, sem0)

        wait_chunk(1, 1, p, sem1)
        compute_chunk(i, 1, 1, p)

        @pl.when(i > 0)
        def _drain_out():
            pltpu.make_async_copy(lg_v.at[pn], out_hbm.at[base + i - 1],
                                  sem_out).wait()

        pltpu.async_copy(lg_v.at[p], out_hbm.at[base + i], sem_out)

        @pl.when(not_last)
        def _wait_rs():
            pltpu.make_async_copy(rs_hbm.at[base + i + 1], rs_v.at[pn],
                                  sem_rs).wait()

        return carry

    lax.fori_loop(0, ROWS_W, row_body, 0)
    pltpu.make_async_copy(lg_v.at[1], out_hbm.at[base + ROWS_W - 1],
                          sem_out).wait()


@functools.cache
def _logits_sc():
    return pl.kernel(
        _logits_sc_body,
        out_type=jax.ShapeDtypeStruct((B, A), jnp.float32),
        mesh=_sc_mesh(),
        scratch_types=[
            pltpu.VMEM((2, 2, CHUNK), jnp.int32),
            pltpu.VMEM((2, A), jnp.int32),
            pltpu.VMEM((ROWS_W, RPAD), jnp.float32),
            pltpu.VMEM((ROWS_W, XPAD), jnp.float32),
            pltpu.VMEM((2, CHUNK, XPAD), jnp.float32),
            pltpu.VMEM((2, A), jnp.float32),
            pltpu.SemaphoreType.DMA,
            pltpu.SemaphoreType.DMA,
            pltpu.SemaphoreType.DMA,
            pltpu.SemaphoreType.DMA,
            pltpu.SemaphoreType.DMA,
        ],
        compiler_params=_SC_PARAMS,
    )


def _softmax_body(lg_ref, mask_ref, p_ref, ent_ref):
    l = lg_ref[...] - (1.0 - mask_ref[...]) * HUGE
    m = jnp.max(l, axis=1, keepdims=True)
    e = jnp.exp(l - m)
    s = jnp.sum(e, axis=1, keepdims=True)
    p = e / s
    p_ref[...] = p
    ent_ref[...] = -jnp.sum(p * jnp.log(p + 1e-20), axis=1, keepdims=True)


def _softmax_entropy(logits, mask):
    nblk = B // _BB
    spec = pl.BlockSpec((_BB, A), lambda i: (i, 0))
    return pl.pallas_call(
        _softmax_body,
        grid=(nblk,),
        in_specs=[spec, spec],
        out_specs=[spec, pl.BlockSpec((_BB, 1), lambda i: (i, 0))],
        out_shape=[jax.ShapeDtypeStruct((B, A), jnp.float32),
                   jax.ShapeDtypeStruct((B, 1), jnp.float32)],
    )(logits, mask)


def kernel(current_entity, source_entity, query_relation, encoded_history,
           r_space, e_space, action_mask, entity_emb, relation_emb,
           W1, b1, W2, b2):
    f32 = jnp.float32
    ce = current_entity.astype(jnp.int32)
    qr3 = query_relation.astype(jnp.int32).reshape(B // _BB, 1, _BB)
    es = e_space.astype(jnp.int32)
    rs = r_space.astype(jnp.int32)

    ta = jnp.pad(entity_emb, ((0, 0), (0, XPAD - D)))
    EA = _gather_e()(ta, ce)

    relp = jnp.zeros((RPAD, D), f32).at[:NUM_RELATIONS].set(relation_emb)
    relt = relp.T
    w1e = jnp.zeros((XPAD, 2 * D), f32).at[:D].set(W1[:D])
    w1h = W1[D:2 * D]
    w1q = W1[2 * D:]
    w2a = W2[:, :D]
    w2e = jnp.zeros((2 * D, XPAD), f32).at[:, :D].set(W2[:, D:])
    b2a = b2[:D].reshape(1, -1)
    b2e = jnp.zeros((1, XPAD), f32).at[0, :D].set(b2[D:])

    X2e, RL = _policy(EA, encoded_history, qr3, w1e, w1h, w1q,
                      b1.reshape(1, -1), relp, w2a, w2e, b2a, b2e, relt)

    logits = _logits_sc()(ta, es, rs, RL, X2e)

    p, ent = _softmax_entropy(logits, action_mask)
    return (p, ent.reshape(-1))


# speedup vs baseline: 1.4788x; 1.4788x over previous
import functools

import jax
import jax.numpy as jnp
from jax import lax
from jax.experimental import pallas as pl
from jax.experimental.pallas import tpu as pltpu
from jax.experimental.pallas import tpu_sc as plsc

NUM_ENTITIES = 100000
NUM_RELATIONS = 474
D = 200
B = 2048
A = 256
RPAD = 512
XPAD = 256
HUGE = 1e31

NC, NS = 2, 16
NW = NC * NS
ROWS_W = B // NW
CHUNK = 128
_BB = 256

_sc_mesh = functools.partial(
    plsc.VectorSubcoreMesh, core_axis_name="c", subcore_axis_name="s",
    num_cores=NC, num_subcores=NS)

_SC_PARAMS = pltpu.CompilerParams(use_tc_tiling_on_sc=False,
                                  needs_layout_passes=False)


def _wid():
    return lax.axis_index("s") * NC + lax.axis_index("c")


_XPERM = tuple(32 * jj + 2 * t + s
               for jj in range(XPAD // 32) for s in (0, 1) for t in range(16))


def _repack_body(x_ref, a_ref, b_ref):
    bf16 = jnp.bfloat16
    a_ref[...] = x_ref[:, :128].astype(bf16)
    tail = x_ref[:, 128:D].astype(bf16)
    b_ref[...] = jnp.concatenate(
        [tail, jnp.zeros((tail.shape[0], 128 - (D - 128)), bf16)], axis=1)


_RB = 2000


def _repack(ent):
    nblk = NUM_ENTITIES // _RB
    return pl.pallas_call(
        _repack_body,
        grid=(nblk,),
        in_specs=[pl.BlockSpec((_RB, D), lambda i: (i, 0))],
        out_specs=[pl.BlockSpec((_RB, 128), lambda i: (i, 0)),
                   pl.BlockSpec((_RB, 128), lambda i: (i, 0))],
        out_shape=[jax.ShapeDtypeStruct((NUM_ENTITIES, 128), jnp.bfloat16),
                   jax.ShapeDtypeStruct((NUM_ENTITIES, 128), jnp.bfloat16)],
    )(ent)


def _gather_e_body(ta_hbm, tb_hbm, ce_hbm, ea_out, eb_out,
                   eidx, arows, brows, sem_a, sem_b):
    base = _wid() * ROWS_W
    pltpu.sync_copy(ce_hbm.at[pl.ds(base, ROWS_W)], eidx)
    cpa = pltpu.async_copy(ta_hbm.at[eidx], arows, sem_a)
    cpb = pltpu.async_copy(tb_hbm.at[eidx], brows, sem_b)
    cpa.wait()
    cpb.wait()
    pltpu.sync_copy(arows, ea_out.at[pl.ds(base, ROWS_W)])
    pltpu.sync_copy(brows, eb_out.at[pl.ds(base, ROWS_W)])


@functools.cache
def _gather_e():
    return pl.kernel(
        _gather_e_body,
        out_type=[jax.ShapeDtypeStruct((B, 128), jnp.bfloat16),
                  jax.ShapeDtypeStruct((B, 128), jnp.bfloat16)],
        mesh=_sc_mesh(),
        scratch_types=[
            pltpu.VMEM((ROWS_W,), jnp.int32),
            pltpu.VMEM((ROWS_W, 128), jnp.bfloat16),
            pltpu.VMEM((ROWS_W, 128), jnp.bfloat16),
            pltpu.SemaphoreType.DMA,
            pltpu.SemaphoreType.DMA,
        ],
        compiler_params=_SC_PARAMS,
    )


def _policy_body(ea_ref, eb_ref, h_ref, qr_ref, w1a_ref, w1b_ref, w1h_ref,
                 w1q_ref, b1_ref, relp_ref, w2a_ref, w2e_ref, b2a_ref,
                 b2e_ref, relt_ref, x2e_ref, rl_ref):
    f32 = jnp.float32
    onehot_t = (lax.broadcasted_iota(jnp.int32, (RPAD, _BB), 0)
                == qr_ref[0]).astype(f32)
    q = lax.dot_general(onehot_t, relp_ref[...], (((0,), (0,)), ((), ())),
                        preferred_element_type=f32)
    h1 = (jnp.dot(ea_ref[...].astype(f32), w1a_ref[...],
                  preferred_element_type=f32)
          + jnp.dot(eb_ref[...].astype(f32), w1b_ref[...],
                    preferred_element_type=f32)
          + jnp.dot(h_ref[...], w1h_ref[...], preferred_element_type=f32)
          + jnp.dot(q, w1q_ref[...], preferred_element_type=f32)
          + b1_ref[...])
    h1 = jnp.maximum(h1, 0.0)
    x2a = jnp.dot(h1, w2a_ref[...], preferred_element_type=f32) + b2a_ref[...]
    x2e = jnp.dot(h1, w2e_ref[...], preferred_element_type=f32) + b2e_ref[...]
    x2e_ref[...] = x2e
    rl_ref[...] = jnp.dot(x2a, relt_ref[...], preferred_element_type=f32)


def _policy(ea, eb, h, qr3, w1a, w1b, w1h, w1q, b1, relp, w2a, w2e,
            b2a, b2e, relt):
    nblk = B // _BB
    row_spec = lambda cols: pl.BlockSpec((_BB, cols), lambda i: (i, 0))
    full = lambda shape: pl.BlockSpec(shape, lambda i: tuple(0 for _ in shape))
    return pl.pallas_call(
        _policy_body,
        grid=(nblk,),
        in_specs=[row_spec(128), row_spec(128), row_spec(D),
                  pl.BlockSpec((1, 1, _BB), lambda i: (i, 0, 0)),
                  full((128, 2 * D)), full((128, 2 * D)), full((D, 2 * D)),
                  full((D, 2 * D)), full((1, 2 * D)), full((RPAD, D)),
                  full((2 * D, D)), full((2 * D, XPAD)),
                  full((1, D)), full((1, XPAD)), full((D, RPAD))],
        out_specs=[row_spec(XPAD), row_spec(RPAD)],
        out_shape=[jax.ShapeDtypeStruct((B, XPAD), jnp.float32),
                   jax.ShapeDtypeStruct((B, RPAD), jnp.float32)],
    )(ea, eb, h, qr3, w1a, w1b, w1h, w1q, b1, relp, w2a, w2e, b2a, b2e, relt)


def _logits_sc_body(ta_hbm, tb_hbm, es_hbm, rs_hbm, rl_hbm, x2_hbm, out_hbm,
                    es_v, rs_v, rl_v, x_v, rows0_v, rows1_v, lg_v,
                    sem0, sem1, sem_es, sem_rs, sem_out):
    base = _wid() * ROWS_W
    pltpu.sync_copy(rl_hbm.at[pl.ds(base, ROWS_W)], rl_v)
    pltpu.sync_copy(x2_hbm.at[pl.ds(base, ROWS_W)], x_v)
    lane = lax.iota(jnp.int32, 16)
    zero = jnp.zeros((16,), jnp.float32)

    def copy_es(i, p, sem=None):
        for c in range(2):
            src = es_hbm.at[base + i, pl.ds(c * CHUNK, CHUNK)]
            if sem is None:
                pltpu.sync_copy(src, es_v.at[p, c])
            else:
                pltpu.async_copy(src, es_v.at[p, c], sem)

    def wait_es(i, p, sem):
        for c in range(2):
            src = es_hbm.at[base + i, pl.ds(c * CHUNK, CHUNK)]
            pltpu.make_async_copy(src, es_v.at[p, c], sem).wait()

    def gather_chunk(c, slot, p, sem):
        idx = es_v.at[p, c]
        pltpu.async_copy(ta_hbm.at[idx], rows0_v.at[slot], sem)
        pltpu.async_copy(tb_hbm.at[idx], rows1_v.at[slot], sem)

    def wait_chunk(c, slot, p, sem):
        idx = es_v.at[p, c]
        pltpu.make_async_copy(ta_hbm.at[idx], rows0_v.at[slot], sem).wait()
        pltpu.make_async_copy(tb_hbm.at[idx], rows1_v.at[slot], sem).wait()

    _gd = lax.GatherDimensionNumbers(offset_dims=(), collapsed_slice_dims=(0,),
                                     start_index_map=(0,))

    def perm_xor(v, s):
        return lax.gather(v, (lane ^ (1 << s))[:, None], _gd, (1,),
                          mode=lax.GatherScatterMode.PROMISE_IN_BOUNDS)

    def reduce16(vs):
        for s in range(4):
            m = (lane & (1 << s)) != 0
            nxt = []
            for k in range(len(vs) // 2):
                a, b = vs[2 * k], vs[2 * k + 1]
                t1 = jnp.where(m, b, a)
                t2 = jnp.where(m, a, b)
                nxt.append(t1 + perm_xor(t2, s))
            vs = nxt
        return vs[0]

    def compute_chunk(i, c, slot, p):
        r0 = rows0_v.at[slot]
        r1 = rows1_v.at[slot]
        ivec = jnp.full((16,), i, jnp.int32)
        xs = [x_v[i, pl.ds(16 * j, 16)] for j in range(14)]

        def blk_body(blk, carry2):
            partials = []
            for t in range(16):
                acc = zero
                for jj in range(4):
                    v = r0[blk * 16 + t, pl.ds(32 * jj, 32)]
                    ve, vo = plsc.unpack(v, format=plsc.PackFormat.INTERLEAVED)
                    acc = acc + ve * xs[2 * jj] + vo * xs[2 * jj + 1]
                for g in range(3):
                    v = r1[blk * 16 + t, pl.ds(32 * g, 32)]
                    ve, vo = plsc.unpack(v, format=plsc.PackFormat.INTERLEAVED)
                    acc = acc + ve * xs[8 + 2 * g] + vo * xs[9 + 2 * g]
                partials.append(acc)
            acc = reduce16(partials)
            ridx = rs_v[p, pl.ds(c * CHUNK + blk * 16, 16)]
            rvals = plsc.load_gather(rl_v, [ivec, ridx])
            lg_v[p, pl.ds(c * CHUNK + blk * 16, 16)] = acc + rvals
            return carry2

        lax.fori_loop(0, CHUNK // 16, blk_body, 0)

    copy_es(0, 0)
    pltpu.sync_copy(rs_hbm.at[base], rs_v.at[0])
    gather_chunk(0, 0, 0, sem0)

    def row_body(i, carry):
        p = lax.rem(i, 2)
        pn = lax.rem(i + 1, 2)
        not_last = i < ROWS_W - 1
        gather_chunk(1, 1, p, sem1)

        @pl.when(not_last)
        def _prefetch_meta():
            copy_es(i + 1, pn, sem_es)
            pltpu.async_copy(rs_hbm.at[base + i + 1], rs_v.at[pn], sem_rs)

        wait_chunk(0, 0, p, sem0)
        compute_chunk(i, 0, 0, p)

        @pl.when(not_last)
        def _issue_next_row():
            wait_es(i + 1, pn, sem_es)
            gather_chunk(0, 0, pn, sem0)

        wait_chunk(1, 1, p, sem1)
        compute_chunk(i, 1, 1, p)

        @pl.when(i > 0)
        def _drain_out():
            pltpu.make_async_copy(lg_v.at[pn], out_hbm.at[base + i - 1],
                                  sem_out).wait()

        pltpu.async_copy(lg_v.at[p], out_hbm.at[base + i], sem_out)

        @pl.when(not_last)
        def _wait_rs():
            pltpu.make_async_copy(rs_hbm.at[base + i + 1], rs_v.at[pn],
                                  sem_rs).wait()

        return carry

    lax.fori_loop(0, ROWS_W, row_body, 0)
    pltpu.make_async_copy(lg_v.at[1], out_hbm.at[base + ROWS_W - 1],
                          sem_out).wait()


@functools.cache
def _logits_sc():
    return pl.kernel(
        _logits_sc_body,
        out_type=jax.ShapeDtypeStruct((B, A), jnp.float32),
        mesh=_sc_mesh(),
        scratch_types=[
            pltpu.VMEM((2, 2, CHUNK), jnp.int32),
            pltpu.VMEM((2, A), jnp.int32),
            pltpu.VMEM((ROWS_W, RPAD), jnp.float32),
            pltpu.VMEM((ROWS_W, XPAD), jnp.float32),
            pltpu.VMEM((2, CHUNK, 128), jnp.bfloat16),
            pltpu.VMEM((2, CHUNK, 128), jnp.bfloat16),
            pltpu.VMEM((2, A), jnp.float32),
            pltpu.SemaphoreType.DMA,
            pltpu.SemaphoreType.DMA,
            pltpu.SemaphoreType.DMA,
            pltpu.SemaphoreType.DMA,
            pltpu.SemaphoreType.DMA,
        ],
        compiler_params=_SC_PARAMS,
    )


def _softmax_body(lg_ref, mask_ref, p_ref, ent_ref):
    l = lg_ref[...] - (1.0 - mask_ref[...]) * HUGE
    m = jnp.max(l, axis=1, keepdims=True)
    e = jnp.exp(l - m)
    s = jnp.sum(e, axis=1, keepdims=True)
    p = e / s
    p_ref[...] = p
    ent_ref[...] = -jnp.sum(p * jnp.log(p + 1e-20), axis=1, keepdims=True)


def _softmax_entropy(logits, mask):
    nblk = B // _BB
    spec = pl.BlockSpec((_BB, A), lambda i: (i, 0))
    return pl.pallas_call(
        _softmax_body,
        grid=(nblk,),
        in_specs=[spec, spec],
        out_specs=[spec, pl.BlockSpec((_BB, 1), lambda i: (i, 0))],
        out_shape=[jax.ShapeDtypeStruct((B, A), jnp.float32),
                   jax.ShapeDtypeStruct((B, 1), jnp.float32)],
    )(logits, mask)


def kernel(current_entity, source_entity, query_relation, encoded_history,
           r_space, e_space, action_mask, entity_emb, relation_emb,
           W1, b1, W2, b2):
    f32 = jnp.float32
    ce = current_entity.astype(jnp.int32)
    qr3 = query_relation.astype(jnp.int32).reshape(B // _BB, 1, _BB)
    es = e_space.astype(jnp.int32)
    rs = r_space.astype(jnp.int32)

    ta, tb = _repack(entity_emb)
    EA, EB = _gather_e()(ta, tb, ce)

    relp = jnp.zeros((RPAD, D), f32).at[:NUM_RELATIONS].set(relation_emb)
    relt = relp.T
    w1a = W1[:128]
    w1b = jnp.zeros((128, 2 * D), f32).at[:D - 128].set(W1[128:D])
    w1h = W1[D:2 * D]
    w1q = W1[2 * D:]
    w2a = W2[:, :D]
    xperm = jnp.array(_XPERM, dtype=jnp.int32)
    w2e = jnp.take(jnp.zeros((2 * D, XPAD), f32).at[:, :D].set(W2[:, D:]),
                   xperm, axis=1)
    b2a = b2[:D].reshape(1, -1)
    b2e = jnp.take(jnp.zeros((1, XPAD), f32).at[0, :D].set(b2[D:]),
                   xperm, axis=1)

    X2e, RL = _policy(EA, EB, encoded_history, qr3, w1a, w1b, w1h, w1q,
                      b1.reshape(1, -1), relp, w2a, w2e, b2a, b2e, relt)

    logits = _logits_sc()(ta, tb, es, rs, RL, X2e)

    p, ent = _softmax_entropy(logits, action_mask)
    return (p, ent.reshape(-1))


# speedup vs baseline: 1.6227x; 1.0973x over previous
import functools

import jax
import jax.numpy as jnp
from jax import lax
from jax.experimental import pallas as pl
from jax.experimental.pallas import tpu as pltpu
from jax.experimental.pallas import tpu_sc as plsc

NUM_ENTITIES = 100000
NUM_RELATIONS = 474
D = 200
B = 2048
A = 256
RPAD = 512
XPAD = 256
HUGE = 1e31

NC, NS = 2, 16
NW = NC * NS
ROWS_W = B // NW
CHUNK = 128
_BB = 256

_sc_mesh = functools.partial(
    plsc.VectorSubcoreMesh, core_axis_name="c", subcore_axis_name="s",
    num_cores=NC, num_subcores=NS)

_SC_PARAMS = pltpu.CompilerParams(use_tc_tiling_on_sc=False,
                                  needs_layout_passes=False)


def _wid():
    return lax.axis_index("s") * NC + lax.axis_index("c")


def _repack_body(x_ref, a_ref):
    a_ref[...] = jnp.concatenate(
        [x_ref[...], jnp.zeros((x_ref.shape[0], XPAD - D), jnp.float32)],
        axis=1)


_RB = 2000


def _repack(ent):
    nblk = NUM_ENTITIES // _RB
    return pl.pallas_call(
        _repack_body,
        grid=(nblk,),
        in_specs=[pl.BlockSpec((_RB, D), lambda i: (i, 0))],
        out_specs=pl.BlockSpec((_RB, XPAD), lambda i: (i, 0)),
        out_shape=jax.ShapeDtypeStruct((NUM_ENTITIES, XPAD), jnp.float32),
    )(ent)


def _gather_e_body(ta_hbm, ce_hbm, ea_out, eidx, arows, sem_a):
    base = _wid() * ROWS_W
    pltpu.sync_copy(ce_hbm.at[pl.ds(base, ROWS_W)], eidx)
    pltpu.async_copy(ta_hbm.at[eidx], arows, sem_a).wait()
    pltpu.sync_copy(arows, ea_out.at[pl.ds(base, ROWS_W)])


@functools.cache
def _gather_e():
    return pl.kernel(
        _gather_e_body,
        out_type=jax.ShapeDtypeStruct((B, XPAD), jnp.float32),
        mesh=_sc_mesh(),
        scratch_types=[
            pltpu.VMEM((ROWS_W,), jnp.int32),
            pltpu.VMEM((ROWS_W, XPAD), jnp.float32),
            pltpu.SemaphoreType.DMA,
        ],
        compiler_params=_SC_PARAMS,
    )


def _policy_body(ea_ref, h_ref, qr_ref, w1e_ref, w1h_ref,
                 w1q_ref, b1_ref, relp_ref, w2a_ref, w2e_ref, b2a_ref,
                 b2e_ref, relt_ref, x2e_ref, rl_ref):
    f32 = jnp.float32
    onehot_t = (lax.broadcasted_iota(jnp.int32, (RPAD, _BB), 0)
                == qr_ref[0]).astype(f32)
    q = lax.dot_general(onehot_t, relp_ref[...], (((0,), (0,)), ((), ())),
                        preferred_element_type=f32)
    h1 = (jnp.dot(ea_ref[...], w1e_ref[...], preferred_element_type=f32)
          + jnp.dot(h_ref[...], w1h_ref[...], preferred_element_type=f32)
          + jnp.dot(q, w1q_ref[...], preferred_element_type=f32)
          + b1_ref[...])
    h1 = jnp.maximum(h1, 0.0)
    x2a = jnp.dot(h1, w2a_ref[...], preferred_element_type=f32) + b2a_ref[...]
    x2e = jnp.dot(h1, w2e_ref[...], preferred_element_type=f32) + b2e_ref[...]
    x2e_ref[...] = x2e
    rl_ref[...] = jnp.dot(x2a, relt_ref[...], preferred_element_type=f32)


def _policy(ea, h, qr3, w1e, w1h, w1q, b1, relp, w2a, w2e, b2a, b2e, relt):
    nblk = B // _BB
    row_spec = lambda cols: pl.BlockSpec((_BB, cols), lambda i: (i, 0))
    full = lambda shape: pl.BlockSpec(shape, lambda i: tuple(0 for _ in shape))
    return pl.pallas_call(
        _policy_body,
        grid=(nblk,),
        in_specs=[row_spec(XPAD), row_spec(D),
                  pl.BlockSpec((1, 1, _BB), lambda i: (i, 0, 0)),
                  full((XPAD, 2 * D)), full((D, 2 * D)),
                  full((D, 2 * D)), full((1, 2 * D)), full((RPAD, D)),
                  full((2 * D, D)), full((2 * D, XPAD)),
                  full((1, D)), full((1, XPAD)), full((D, RPAD))],
        out_specs=[row_spec(XPAD), row_spec(RPAD)],
        out_shape=[jax.ShapeDtypeStruct((B, XPAD), jnp.float32),
                   jax.ShapeDtypeStruct((B, RPAD), jnp.float32)],
    )(ea, h, qr3, w1e, w1h, w1q, b1, relp, w2a, w2e, b2a, b2e, relt)


def _logits_sc_body(ta_hbm, es_hbm, rs_hbm, rl_hbm, x2_hbm, out_hbm,
                    es_v, rs_v, rl_v, x_v, rows0_v, lg_v,
                    sem0, sem1, sem_es, sem_rs, sem_out):
    base = _wid() * ROWS_W
    pltpu.sync_copy(rl_hbm.at[pl.ds(base, ROWS_W)], rl_v)
    pltpu.sync_copy(x2_hbm.at[pl.ds(base, ROWS_W)], x_v)
    lane = lax.iota(jnp.int32, 16)
    zero = jnp.zeros((16,), jnp.float32)

    def copy_es(i, p, sem=None):
        for c in range(2):
            src = es_hbm.at[base + i, pl.ds(c * CHUNK, CHUNK)]
            if sem is None:
                pltpu.sync_copy(src, es_v.at[p, c])
            else:
                pltpu.async_copy(src, es_v.at[p, c], sem)

    def wait_es(i, p, sem):
        for c in range(2):
            src = es_hbm.at[base + i, pl.ds(c * CHUNK, CHUNK)]
            pltpu.make_async_copy(src, es_v.at[p, c], sem).wait()

    def gather_chunk(c, slot, p, sem):
        idx = es_v.at[p, c]
        return pltpu.async_copy(ta_hbm.at[idx], rows0_v.at[slot], sem)

    def wait_chunk(c, slot, p, sem):
        idx = es_v.at[p, c]
        pltpu.make_async_copy(ta_hbm.at[idx], rows0_v.at[slot], sem).wait()

    _gd = lax.GatherDimensionNumbers(offset_dims=(), collapsed_slice_dims=(0,),
                                     start_index_map=(0,))

    def perm_xor(v, s):
        return lax.gather(v, (lane ^ (1 << s))[:, None], _gd, (1,),
                          mode=lax.GatherScatterMode.PROMISE_IN_BOUNDS)

    def reduce16(vs):
        for s in range(4):
            m = (lane & (1 << s)) != 0
            nxt = []
            for k in range(len(vs) // 2):
                a, b = vs[2 * k], vs[2 * k + 1]
                t1 = jnp.where(m, b, a)
                t2 = jnp.where(m, a, b)
                nxt.append(t1 + perm_xor(t2, s))
            vs = nxt
        return vs[0]

    def compute_chunk(i, c, slot, p):
        r0 = rows0_v.at[slot]
        ivec = jnp.full((16,), i, jnp.int32)
        xs = [x_v[i, pl.ds(16 * j, 16)] for j in range(13)]

        def blk_body(blk, carry2):
            partials = []
            for t in range(16):
                acc = zero
                for j in range(13):
                    v = r0[blk * 16 + t, pl.ds(16 * j, 16)]
                    acc = acc + v * xs[j]
                partials.append(acc)
            acc = reduce16(partials)
            ridx = rs_v[p, pl.ds(c * CHUNK + blk * 16, 16)]
            rvals = plsc.load_gather(rl_v, [ivec, ridx])
            lg_v[p, pl.ds(c * CHUNK + blk * 16, 16)] = acc + rvals
            return carry2

        lax.fori_loop(0, CHUNK // 16, blk_body, 0)

    copy_es(0, 0)
    pltpu.sync_copy(rs_hbm.at[base], rs_v.at[0])
    gather_chunk(0, 0, 0, sem0)

    def row_body(i, carry):
        p = lax.rem(i, 2)
        pn = lax.rem(i + 1, 2)
        not_last = i < ROWS_W - 1
        gather_chunk(1, 1, p, sem1)

        @pl.when(not_last)
        def _prefetch_meta():
            copy_es(i + 1, pn, sem_es)
            pltpu.async_copy(rs_hbm.at[base + i + 1], rs_v.at[pn], sem_rs)

        wait_chunk(0, 0, p, sem0)
        compute_chunk(i, 0, 0, p)

        @pl.when(not_last)
        def _issue_next_row():
            wait_es(i + 1, pn, sem_es)
            gather_chunk(0, 0, pn, sem0)

        wait_chunk(1, 1, p, sem1)
        compute_chunk(i, 1, 1, p)

        @pl.when(i > 0)
        def _drain_out():
            pltpu.make_async_copy(lg_v.at[pn], out_hbm.at[base + i - 1],
                                  sem_out).wait()

        pltpu.async_copy(lg_v.at[p], out_hbm.at[base + i], sem_out)

        @pl.when(not_last)
        def _wait_rs():
            pltpu.make_async_copy(rs_hbm.at[base + i + 1], rs_v.at[pn],
                                  sem_rs).wait()

        return carry

    lax.fori_loop(0, ROWS_W, row_body, 0)
    pltpu.make_async_copy(lg_v.at[1], out_hbm.at[base + ROWS_W - 1],
                          sem_out).wait()


@functools.cache
def _logits_sc():
    return pl.kernel(
        _logits_sc_body,
        out_type=jax.ShapeDtypeStruct((B, A), jnp.float32),
        mesh=_sc_mesh(),
        scratch_types=[
            pltpu.VMEM((2, 2, CHUNK), jnp.int32),
            pltpu.VMEM((2, A), jnp.int32),
            pltpu.VMEM((ROWS_W, RPAD), jnp.float32),
            pltpu.VMEM((ROWS_W, XPAD), jnp.float32),
            pltpu.VMEM((2, CHUNK, XPAD), jnp.float32),
            pltpu.VMEM((2, A), jnp.float32),
            pltpu.SemaphoreType.DMA,
            pltpu.SemaphoreType.DMA,
            pltpu.SemaphoreType.DMA,
            pltpu.SemaphoreType.DMA,
            pltpu.SemaphoreType.DMA,
        ],
        compiler_params=_SC_PARAMS,
    )


def _softmax_body(lg_ref, mask_ref, p_ref, ent_ref):
    l = lg_ref[...] - (1.0 - mask_ref[...]) * HUGE
    m = jnp.max(l, axis=1, keepdims=True)
    e = jnp.exp(l - m)
    s = jnp.sum(e, axis=1, keepdims=True)
    p = e / s
    p_ref[...] = p
    ent_ref[...] = -jnp.sum(p * jnp.log(p + 1e-20), axis=1, keepdims=True)


def _softmax_entropy(logits, mask):
    nblk = B // _BB
    spec = pl.BlockSpec((_BB, A), lambda i: (i, 0))
    return pl.pallas_call(
        _softmax_body,
        grid=(nblk,),
        in_specs=[spec, spec],
        out_specs=[spec, pl.BlockSpec((_BB, 1), lambda i: (i, 0))],
        out_shape=[jax.ShapeDtypeStruct((B, A), jnp.float32),
                   jax.ShapeDtypeStruct((B, 1), jnp.float32)],
    )(logits, mask)


def kernel(current_entity, source_entity, query_relation, encoded_history,
           r_space, e_space, action_mask, entity_emb, relation_emb,
           W1, b1, W2, b2):
    f32 = jnp.float32
    ce = current_entity.astype(jnp.int32)
    qr3 = query_relation.astype(jnp.int32).reshape(B // _BB, 1, _BB)
    es = e_space.astype(jnp.int32)
    rs = r_space.astype(jnp.int32)

    ta = _repack(entity_emb)
    EA = _gather_e()(ta, ce)

    relp = jnp.zeros((RPAD, D), f32).at[:NUM_RELATIONS].set(relation_emb)
    relt = relp.T
    w1e = jnp.zeros((XPAD, 2 * D), f32).at[:D].set(W1[:D])
    w1h = W1[D:2 * D]
    w1q = W1[2 * D:]
    w2a = W2[:, :D]
    w2e = jnp.zeros((2 * D, XPAD), f32).at[:, :D].set(W2[:, D:])
    b2a = b2[:D].reshape(1, -1)
    b2e = jnp.zeros((1, XPAD), f32).at[0, :D].set(b2[D:])

    X2e, RL = _policy(EA, encoded_history, qr3, w1e, w1h, w1q,
                      b1.reshape(1, -1), relp, w2a, w2e, b2a, b2e, relt)

    logits = _logits_sc()(ta, es, rs, RL, X2e)

    p, ent = _softmax_entropy(logits, action_mask)
    return (p, ent.reshape(-1))
